# Initial kernel scaffold; baseline (speedup 1.0000x reference)
import jax
import jax.numpy as jnp
from jax.experimental import pallas as pl


def kernel(X, edge_index, edge_weight, W1, b1, W2, b2, Wl, bl):
    raise NotImplementedError("write your pallas kernel here")


# speedup vs baseline: 8.6509x; 8.6509x over previous
import functools

import jax
import jax.numpy as jnp
from jax import lax
from jax.experimental import pallas as pl
from jax.experimental.pallas import tpu as pltpu
from jax.experimental.pallas import tpu_sc as plsc

N = 10000
E = 320000
D_IN = 128
H = 64

NC = 2
NS = 16
NW = NC * NS
EPW = E // NW
CH = 80
NCHUNK = EPW // CH

N_PAD = 10240
DPW = N_PAD // NS

N_ACC = 10240
RPT = N_ACC // NS
ZR = 128

_MESH = plsc.VectorSubcoreMesh(core_axis_name="c", subcore_axis_name="s")


def _zero_vec():
    return jnp.zeros((16,), jnp.float32)


@functools.partial(
    pl.kernel,
    out_type=jax.ShapeDtypeStruct((NC, N_PAD), jnp.float32),
    mesh=_MESH,
    scratch_types=[
        pltpu.VMEM((CH,), jnp.int32),
        pltpu.VMEM((CH,), jnp.float32),
        pltpu.VMEM((DPW,), jnp.float32),
        pltpu.VMEM_SHARED((N_PAD,), jnp.float32),
    ],
)
def _deg_kernel(dst_hbm, w_hbm, out_hbm, idx_v, w_v, zbuf, deg_sh):
    c = lax.axis_index("c")
    s = lax.axis_index("s")
    wid = c * NS + s

    def zfill(j, _):
        zbuf[pl.ds(j * 16, 16)] = _zero_vec()
        return 0

    lax.fori_loop(0, DPW // 16, zfill, 0)
    pltpu.sync_copy(zbuf, deg_sh.at[pl.ds(s * DPW, DPW)])
    plsc.subcore_barrier()

    def chunk(i, _):
        base = wid * EPW + i * CH
        pltpu.sync_copy(dst_hbm.at[pl.ds(base, CH)], idx_v)
        pltpu.sync_copy(w_hbm.at[pl.ds(base, CH)], w_v)
        pltpu.sync_copy(w_v, deg_sh.at[idx_v], add=True)
        return 0

    lax.fori_loop(0, NCHUNK, chunk, 0)
    plsc.subcore_barrier()
    pltpu.sync_copy(deg_sh.at[pl.ds(s * DPW, DPW)],
                    out_hbm.at[c, pl.ds(s * DPW, DPW)])


@functools.partial(
    pl.kernel,
    out_type=jax.ShapeDtypeStruct((NC, N_ACC, H), jnp.float32),
    mesh=_MESH,
    scratch_types=[
        pltpu.VMEM((CH,), jnp.int32),
        pltpu.VMEM((CH,), jnp.int32),
        pltpu.VMEM((CH,), jnp.float32),
        pltpu.VMEM((CH, H), jnp.float32),
        pltpu.VMEM((ZR, H), jnp.float32),
        pltpu.VMEM_SHARED((N_ACC, H), jnp.float32),
        pltpu.SemaphoreType.DMA,
    ],
    compiler_params=pltpu.CompilerParams(use_tc_tiling_on_sc=False),
)
def _edge_kernel(src_hbm, dst_hbm, w_hbm, y_hbm, out_hbm,
                 src_v, dst_v, w_v, rows, zbuf, acc_sh, sem):
    c = lax.axis_index("c")
    s = lax.axis_index("s")
    wid = c * NS + s

    def zfill(j, _):
        r = j // (H // 16)
        q = j % (H // 16)
        zbuf[r, pl.ds(q * 16, 16)] = _zero_vec()
        return 0

    lax.fori_loop(0, ZR * (H // 16), zfill, 0)
    for k in range(RPT // ZR):
        pltpu.sync_copy(zbuf, acc_sh.at[pl.ds(s * RPT + k * ZR, ZR), :])
    plsc.subcore_barrier()

    def chunk(i, _):
        base = wid * EPW + i * CH
        pltpu.sync_copy(src_hbm.at[pl.ds(base, CH)], src_v)
        pltpu.sync_copy(dst_hbm.at[pl.ds(base, CH)], dst_v)
        pltpu.sync_copy(w_hbm.at[pl.ds(base, CH)], w_v)
        pltpu.async_copy(y_hbm.at[src_v], rows, sem).wait()

        def scale(g, _):
            wg = w_v[pl.ds(g * 16, 16)]
            for l in range(16):
                wj = wg[l]
                j = g * 16 + l
                for q in range(H // 16):
                    rows[j, pl.ds(q * 16, 16)] = rows[j, pl.ds(q * 16, 16)] * wj
            return 0

        lax.fori_loop(0, CH // 16, scale, 0)
        pltpu.sync_copy(rows, acc_sh.at[dst_v], add=True)
        return 0

    lax.fori_loop(0, NCHUNK, chunk, 0)
    plsc.subcore_barrier()
    pltpu.sync_copy(acc_sh.at[pl.ds(s * RPT, RPT), :],
                    out_hbm.at[c, pl.ds(s * RPT, RPT), :])


_BR = 1000


def _dense1_body(x_ref, w1_ref, degp_ref, y_ref, dis_ref):
    xw = jnp.dot(x_ref[...], w1_ref[...], preferred_element_type=jnp.float32)
    deg = 1.0 + degp_ref[:, 0:1] + degp_ref[:, 1:2]
    dis = lax.rsqrt(deg)
    y_ref[...] = dis * xw
    dis_ref[...] = dis


def _dense2_body(p_ref, y_ref, dis_ref, w2_ref, b1_ref, y2_ref):
    dis = dis_ref[...]
    acc = p_ref[0] + p_ref[1] + y_ref[...]
    h = jnp.maximum(dis * acc + b1_ref[...], 0.0)
    y2_ref[...] = dis * jnp.dot(h, w2_ref[...],
                                preferred_element_type=jnp.float32)


def _dense3_body(p_ref, y_ref, dis_ref, wl_ref, b2_ref, bl_ref, z_ref):
    dis = dis_ref[...]
    acc = p_ref[0] + p_ref[1] + y_ref[...]
    h = jnp.maximum(dis * acc + b2_ref[...], 0.0)
    logit = jnp.dot(h, wl_ref[...], preferred_element_type=jnp.float32)
    z_ref[...] = jax.nn.sigmoid(logit + bl_ref[...])


def _dense1(X, W1, degp):
    return pl.pallas_call(
        _dense1_body,
        grid=(N // _BR,),
        in_specs=[
            pl.BlockSpec((_BR, D_IN), lambda i: (i, 0)),
            pl.BlockSpec((D_IN, H), lambda i: (0, 0)),
            pl.BlockSpec((_BR, NC), lambda i: (i, 0)),
        ],
        out_specs=[
            pl.BlockSpec((_BR, H), lambda i: (i, 0)),
            pl.BlockSpec((_BR, 1), lambda i: (i, 0)),
        ],
        out_shape=[
            jax.ShapeDtypeStruct((N, H), jnp.float32),
            jax.ShapeDtypeStruct((N, 1), jnp.float32),
        ],
    )(X, W1, degp)


def _dense2(p, y, dis, W2, b1):
    return pl.pallas_call(
        _dense2_body,
        grid=(N // _BR,),
        in_specs=[
            pl.BlockSpec((NC, _BR, H), lambda i: (0, i, 0)),
            pl.BlockSpec((_BR, H), lambda i: (i, 0)),
            pl.BlockSpec((_BR, 1), lambda i: (i, 0)),
            pl.BlockSpec((H, H), lambda i: (0, 0)),
            pl.BlockSpec((1, H), lambda i: (0, 0)),
        ],
        out_specs=pl.BlockSpec((_BR, H), lambda i: (i, 0)),
        out_shape=jax.ShapeDtypeStruct((N, H), jnp.float32),
    )(p, y, dis, W2, b1)


def _dense3(p, y, dis, Wl, b2, bl):
    return pl.pallas_call(
        _dense3_body,
        grid=(N // _BR,),
        in_specs=[
            pl.BlockSpec((NC, _BR, H), lambda i: (0, i, 0)),
            pl.BlockSpec((_BR, H), lambda i: (i, 0)),
            pl.BlockSpec((_BR, 1), lambda i: (i, 0)),
            pl.BlockSpec((H, 1), lambda i: (0, 0)),
            pl.BlockSpec((1, H), lambda i: (0, 0)),
            pl.BlockSpec((1, 1), lambda i: (0, 0)),
        ],
        out_specs=pl.BlockSpec((_BR, 1), lambda i: (i, 0)),
        out_shape=jax.ShapeDtypeStruct((N, 1), jnp.float32),
    )(p, y, dis, Wl, b2, bl)


def kernel(X, edge_index, edge_weight, W1, b1, W2, b2, Wl, bl):
    src = edge_index[0]
    dst = edge_index[1]

    degp = _deg_kernel(dst, edge_weight)
    y1, dis = _dense1(X, W1, degp[:, :N].T)

    p1 = _edge_kernel(src, dst, edge_weight, y1)
    y2 = _dense2(p1, y1, dis, W2, b1.reshape(1, H))

    p2 = _edge_kernel(src, dst, edge_weight, y2)
    z = _dense3(p2, y2, dis, Wl, b2.reshape(1, H), bl.reshape(1, 1))
    return z[:, 0]


# speedup vs baseline: 14.8922x; 1.7215x over previous
import functools

import jax
import jax.numpy as jnp
from jax import lax
from jax.experimental import pallas as pl
from jax.experimental.pallas import tpu as pltpu
from jax.experimental.pallas import tpu_sc as plsc

N = 10000
E = 320000
D_IN = 128
H = 64

NC = 2
NS = 16
NW = NC * NS
EPW = E // NW
CH = 80
NCHUNK = EPW // CH

N_PAD = 10240
DPW = N_PAD // NS

N_ACC = 10240
RPT = N_ACC // NS
ZR = 128

_MESH = plsc.VectorSubcoreMesh(core_axis_name="c", subcore_axis_name="s")


def _zero_vec():
    return jnp.zeros((16,), jnp.float32)


@functools.partial(
    pl.kernel,
    out_type=jax.ShapeDtypeStruct((NC, N_PAD), jnp.float32),
    mesh=_MESH,
    scratch_types=[
        pltpu.VMEM((NCHUNK, CH), jnp.int32),
        pltpu.VMEM((NCHUNK, CH), jnp.float32),
        pltpu.VMEM((DPW,), jnp.float32),
        pltpu.VMEM_SHARED((N_PAD,), jnp.float32),
        pltpu.SemaphoreType.DMA,
        pltpu.SemaphoreType.DMA,
    ],
)
def _deg_kernel(dst_hbm, w_hbm, out_hbm, dst2d, w2d, zbuf, deg_sh, sm0, sm1):
    c = lax.axis_index("c")
    s = lax.axis_index("s")
    wid = c * NS + s

    cp0 = pltpu.async_copy(dst_hbm.at[wid], dst2d, sm0)
    cp1 = pltpu.async_copy(w_hbm.at[wid], w2d, sm1)

    def zfill(j, _):
        zbuf[pl.ds(j * 16, 16)] = _zero_vec()
        return 0

    lax.fori_loop(0, DPW // 16, zfill, 0)
    pltpu.sync_copy(zbuf, deg_sh.at[pl.ds(s * DPW, DPW)])
    cp0.wait()
    cp1.wait()
    plsc.subcore_barrier()

    sems = (sm0, sm1)

    def s_issue(i, b):
        pltpu.async_copy(w2d.at[i], deg_sh.at[dst2d.at[i]], sems[b], add=True)

    def s_wait(i, b):
        pltpu.make_async_copy(w2d.at[i], deg_sh.at[dst2d.at[i]],
                              sems[b]).wait()

    s_issue(0, 0)
    s_issue(1, 1)

    def pair(k, _):
        i0 = 2 * k
        s_wait(i0 - 2, 0)
        s_issue(i0, 0)
        s_wait(i0 - 1, 1)
        s_issue(i0 + 1, 1)
        return 0

    lax.fori_loop(1, (NCHUNK - 1) // 2, pair, 0)
    s_wait(NCHUNK - 3, 0)
    s_issue(NCHUNK - 1, 0)
    s_wait(NCHUNK - 2, 1)
    s_wait(NCHUNK - 1, 0)
    plsc.subcore_barrier()
    pltpu.sync_copy(deg_sh.at[pl.ds(s * DPW, DPW)],
                    out_hbm.at[c, pl.ds(s * DPW, DPW)])


@functools.partial(
    pl.kernel,
    out_type=jax.ShapeDtypeStruct((NC, N_ACC, H), jnp.float32),
    mesh=_MESH,
    scratch_types=[
        pltpu.VMEM((NCHUNK, CH), jnp.int32),
        pltpu.VMEM((NCHUNK, CH), jnp.int32),
        pltpu.VMEM((NCHUNK, CH), jnp.float32),
        pltpu.VMEM((CH, H), jnp.float32),
        pltpu.VMEM((CH, H), jnp.float32),
        pltpu.VMEM((ZR, H), jnp.float32),
        pltpu.VMEM_SHARED((N_ACC, H), jnp.float32),
        pltpu.SemaphoreType.DMA,
        pltpu.SemaphoreType.DMA,
        pltpu.SemaphoreType.DMA,
        pltpu.SemaphoreType.DMA,
    ],
    compiler_params=pltpu.CompilerParams(use_tc_tiling_on_sc=False),
)
def _edge_kernel(src_hbm, dst_hbm, w_hbm, y_hbm, out_hbm,
                 src2d, dst2d, w2d, r0, r1, zbuf, acc_sh, g0, g1, s0, s1):
    c = lax.axis_index("c")
    s = lax.axis_index("s")
    wid = c * NS + s
    rows = (r0, r1)
    gsem = (g0, g1)
    ssem = (s0, s1)

    cp0 = pltpu.async_copy(src_hbm.at[wid], src2d, g0)
    cp1 = pltpu.async_copy(dst_hbm.at[wid], dst2d, g1)
    cp2 = pltpu.async_copy(w_hbm.at[wid], w2d, s0)

    def zfill(j, _):
        r = j // (H // 16)
        q = j % (H // 16)
        zbuf[r, pl.ds(q * 16, 16)] = _zero_vec()
        return 0

    lax.fori_loop(0, ZR * (H // 16), zfill, 0)
    for k in range(RPT // ZR):
        pltpu.sync_copy(zbuf, acc_sh.at[pl.ds(s * RPT + k * ZR, ZR), :])
    cp0.wait()
    cp1.wait()
    cp2.wait()
    plsc.subcore_barrier()

    def g_issue(i, b):
        pltpu.async_copy(y_hbm.at[src2d.at[i]], rows[b], gsem[b])

    def g_wait(i, b):
        pltpu.make_async_copy(y_hbm.at[src2d.at[i]], rows[b], gsem[b]).wait()

    def s_issue(i, b):
        pltpu.async_copy(rows[b], acc_sh.at[dst2d.at[i]], ssem[b], add=True)

    def s_wait(i, b):
        pltpu.make_async_copy(rows[b], acc_sh.at[dst2d.at[i]],
                              ssem[b]).wait()

    def scale(i, b):
        rp = rows[b]

        def grp(g, _):
            wg = w2d[i, pl.ds(g * 16, 16)]
            for l in range(16):
                wj = wg[l]
                j = g * 16 + l
                for q in range(H // 16):
                    rp[j, pl.ds(q * 16, 16)] = rp[j, pl.ds(q * 16, 16)] * wj
            return 0

        lax.fori_loop(0, CH // 16, grp, 0)

    def step(i, b):
        q = 1 - b
        g_wait(i, b)
        scale(i, b)
        s_issue(i, b)
        s_wait(i - 1, q)
        g_issue(i + 1, q)

    g_issue(0, 0)
    g_issue(1, 1)
    g_wait(0, 0)
    scale(0, 0)
    s_issue(0, 0)

    def pair(k, _):
        step(2 * k + 1, 1)
        step(2 * k + 2, 0)
        return 0

    lax.fori_loop(0, (NCHUNK - 3) // 2, pair, 0)
    step(NCHUNK - 2, 1)
    g_wait(NCHUNK - 1, 0)
    scale(NCHUNK - 1, 0)
    s_issue(NCHUNK - 1, 0)
    s_wait(NCHUNK - 2, 1)
    s_wait(NCHUNK - 1, 0)
    plsc.subcore_barrier()
    pltpu.sync_copy(acc_sh.at[pl.ds(s * RPT, RPT), :],
                    out_hbm.at[c, pl.ds(s * RPT, RPT), :])


_BR = 1000


def _dense1_body(x_ref, w1_ref, degp_ref, y_ref, dis_ref):
    xw = jnp.dot(x_ref[...], w1_ref[...], preferred_element_type=jnp.float32)
    deg = 1.0 + degp_ref[:, 0:1] + degp_ref[:, 1:2]
    dis = lax.rsqrt(deg)
    y_ref[...] = dis * xw
    dis_ref[...] = dis


def _dense2_body(p_ref, y_ref, dis_ref, w2_ref, b1_ref, y2_ref):
    dis = dis_ref[...]
    acc = p_ref[0] + p_ref[1] + y_ref[...]
    h = jnp.maximum(dis * acc + b1_ref[...], 0.0)
    y2_ref[...] = dis * jnp.dot(h, w2_ref[...],
                                preferred_element_type=jnp.float32)


def _dense3_body(p_ref, y_ref, dis_ref, wl_ref, b2_ref, bl_ref, z_ref):
    dis = dis_ref[...]
    acc = p_ref[0] + p_ref[1] + y_ref[...]
    h = jnp.maximum(dis * acc + b2_ref[...], 0.0)
    logit = jnp.dot(h, wl_ref[...], preferred_element_type=jnp.float32)
    z_ref[...] = jax.nn.sigmoid(logit + bl_ref[...])


def _dense1(X, W1, degp):
    return pl.pallas_call(
        _dense1_body,
        grid=(N // _BR,),
        in_specs=[
            pl.BlockSpec((_BR, D_IN), lambda i: (i, 0)),
            pl.BlockSpec((D_IN, H), lambda i: (0, 0)),
            pl.BlockSpec((_BR, NC), lambda i: (i, 0)),
        ],
        out_specs=[
            pl.BlockSpec((_BR, H), lambda i: (i, 0)),
            pl.BlockSpec((_BR, 1), lambda i: (i, 0)),
        ],
        out_shape=[
            jax.ShapeDtypeStruct((N, H), jnp.float32),
            jax.ShapeDtypeStruct((N, 1), jnp.float32),
        ],
    )(X, W1, degp)


def _dense2(p, y, dis, W2, b1):
    return pl.pallas_call(
        _dense2_body,
        grid=(N // _BR,),
        in_specs=[
            pl.BlockSpec((NC, _BR, H), lambda i: (0, i, 0)),
            pl.BlockSpec((_BR, H), lambda i: (i, 0)),
            pl.BlockSpec((_BR, 1), lambda i: (i, 0)),
            pl.BlockSpec((H, H), lambda i: (0, 0)),
            pl.BlockSpec((1, H), lambda i: (0, 0)),
        ],
        out_specs=pl.BlockSpec((_BR, H), lambda i: (i, 0)),
        out_shape=jax.ShapeDtypeStruct((N, H), jnp.float32),
    )(p, y, dis, W2, b1)


def _dense3(p, y, dis, Wl, b2, bl):
    return pl.pallas_call(
        _dense3_body,
        grid=(N // _BR,),
        in_specs=[
            pl.BlockSpec((NC, _BR, H), lambda i: (0, i, 0)),
            pl.BlockSpec((_BR, H), lambda i: (i, 0)),
            pl.BlockSpec((_BR, 1), lambda i: (i, 0)),
            pl.BlockSpec((H, 1), lambda i: (0, 0)),
            pl.BlockSpec((1, H), lambda i: (0, 0)),
            pl.BlockSpec((1, 1), lambda i: (0, 0)),
        ],
        out_specs=pl.BlockSpec((_BR, 1), lambda i: (i, 0)),
        out_shape=jax.ShapeDtypeStruct((N, 1), jnp.float32),
    )(p, y, dis, Wl, b2, bl)


def kernel(X, edge_index, edge_weight, W1, b1, W2, b2, Wl, bl):
    src = edge_index[0].reshape(NW, NCHUNK, CH)
    dst = edge_index[1].reshape(NW, NCHUNK, CH)
    ew = edge_weight.reshape(NW, NCHUNK, CH)

    degp = _deg_kernel(dst, ew)
    y1, dis = _dense1(X, W1, degp[:, :N].T)

    p1 = _edge_kernel(src, dst, ew, y1)
    y2 = _dense2(p1, y1, dis, W2, b1.reshape(1, H))

    p2 = _edge_kernel(src, dst, ew, y2)
    z = _dense3(p2, y2, dis, Wl, b2.reshape(1, H), bl.reshape(1, 1))
    return z[:, 0]
